# baseline (speedup 1.0000x reference)
import functools

import jax
import jax.numpy as jnp
from jax import lax
from jax.experimental import pallas as pl
from jax.experimental.pallas import tpu as pltpu
from jax.experimental.pallas import tpu_sc as plsc

N = 10000
E = 160000
D = 128
EPS = 1e-5

NC = 2
NS = 16
NW = NC * NS
LANES = 16

EPW = E // NW
GCH = 1000

NPT = 320
NPAD = NPT * NW
BIG = 3.0e38

EB = 4000
NB = 1000


_MESH = dict(core_axis_name="c", subcore_axis_name="s")


def _worker_id():
    return lax.axis_index("s") * NC + lax.axis_index("c")


def _sc_gather(x, src, dst):
    @functools.partial(
        pl.kernel,
        mesh=plsc.VectorSubcoreMesh(**_MESH),
        out_type=[
            jax.ShapeDtypeStruct((E, D), jnp.float32),
            jax.ShapeDtypeStruct((E, D), jnp.float32),
        ],
        scratch_types=[
            pltpu.VMEM((GCH,), jnp.int32),
            pltpu.VMEM((GCH, D), jnp.float32),
            pltpu.SemaphoreType.DMA,
        ],
    )
    def k(x_hbm, src_hbm, dst_hbm, xs_out, xd_out, idx_v, rows_v, sem):
        base = _worker_id() * EPW

        def body(c, carry):
            off = base + c * GCH
            pltpu.sync_copy(src_hbm.at[pl.ds(off, GCH)], idx_v)
            pltpu.async_copy(x_hbm.at[idx_v], rows_v, sem).wait()
            pltpu.sync_copy(rows_v, xs_out.at[pl.ds(off, GCH)])
            pltpu.sync_copy(dst_hbm.at[pl.ds(off, GCH)], idx_v)
            pltpu.async_copy(x_hbm.at[idx_v], rows_v, sem).wait()
            pltpu.sync_copy(rows_v, xd_out.at[pl.ds(off, GCH)])
            return carry

        lax.fori_loop(0, EPW // GCH, body, 0)

    return k(x, src, dst)


FB = 528
FCAP = 5120
LISTSZ = NW * NW * FCAP
SB = 512
BPF = FCAP // SB


def _sc_bucket(dst_pad, ramp):
    @functools.partial(
        pl.kernel,
        mesh=plsc.VectorSubcoreMesh(**_MESH),
        out_type=[
            jax.ShapeDtypeStruct((LISTSZ,), jnp.int32),
            jax.ShapeDtypeStruct((LISTSZ,), jnp.int32),
        ],
        scratch_types=[
            pltpu.VMEM((EPW + 24,), jnp.int32),
            pltpu.VMEM((EPW + 24,), jnp.int32),
            pltpu.VMEM((NW * FB,), jnp.int32),
            pltpu.VMEM((NW * FB,), jnp.int32),
            pltpu.SMEM((NW,), jnp.int32),
            pltpu.SMEM((NW,), jnp.int32),
        ],
    )
    def k(dst_hbm, ramp_hbm, eidlist, dstlist, dstv, rampv, fbe, fbd,
          cnts, flushed):
        wid = _worker_id()
        sbase = wid * EPW
        iota = lax.iota(jnp.int32, LANES)

        def init_cnt(i, carry):
            cnts[i] = jnp.int32(0)
            flushed[i] = jnp.int32(0)
            return carry

        lax.fori_loop(0, NW, init_cnt, 0)

        pltpu.sync_copy(dst_hbm.at[pl.ds(sbase, EPW + 24)], dstv)
        pltpu.sync_copy(ramp_hbm.at[pl.ds(sbase, EPW + 24)], rampv)

        def lane(v, kk):
            d16 = dstv[pl.ds(v * LANES, LANES)]
            dk = d16[kk]
            ow = dk // NPT
            pos = cnts[ow]
            fb0 = ow * FB
            w = pl.ds(v * LANES + kk, LANES)
            fbe[pl.ds(fb0 + pos, LANES)] = rampv[w]
            fbd[pl.ds(fb0 + pos, LANES)] = dstv[w]
            cnts[ow] = pos + 1

            @pl.when(pos + 1 >= 512)
            def _():
                base = pl.multiple_of(
                    (wid * NW + ow) * FCAP + flushed[ow], 8)
                src = pl.multiple_of(fb0, 8)
                pltpu.sync_copy(fbe.at[pl.ds(src, 512)],
                                eidlist.at[pl.ds(base, 512)])
                pltpu.sync_copy(fbd.at[pl.ds(src, 512)],
                                dstlist.at[pl.ds(base, 512)])
                fbe[pl.ds(fb0, LANES)] = fbe[pl.ds(fb0 + 512, LANES)]
                fbd[pl.ds(fb0, LANES)] = fbd[pl.ds(fb0 + 512, LANES)]
                cnts[ow] = pos + 1 - 512
                flushed[ow] = flushed[ow] + 512

        def vreg(v, carry):
            for kk in range(LANES):
                lane(v, kk)
            return carry

        lax.fori_loop(0, EPW // LANES, vreg, 0)
        for kk in range(EPW - (EPW // LANES) * LANES):
            lane(EPW // LANES, kk)

        def finflush(ow, carry):
            p = cnts[ow]
            fb0 = ow * FB
            for i in range(512 // LANES):
                sl = pl.ds(fb0 + i * LANES, LANES)
                keep = (iota + (i * LANES)) < p
                fbe[sl] = jnp.where(keep, fbe[sl], 0)
                fbd[sl] = jnp.where(keep, fbd[sl], -1)
            base = pl.multiple_of((wid * NW + ow) * FCAP + flushed[ow], 8)
            src = pl.multiple_of(fb0, 8)
            pltpu.sync_copy(fbe.at[pl.ds(src, 512)],
                            eidlist.at[pl.ds(base, 512)])
            pltpu.sync_copy(fbd.at[pl.ds(src, 512)],
                            dstlist.at[pl.ds(base, 512)])
            return carry

        lax.fori_loop(0, NW, finflush, 0)

    return k(dst_pad, ramp)


def _sc_scatter_max(h2, eidlist, dstlist):
    @functools.partial(
        pl.kernel,
        mesh=plsc.VectorSubcoreMesh(**_MESH),
        out_type=jax.ShapeDtypeStruct((NPAD, D), jnp.float32),
        scratch_types=[
            pltpu.VMEM((NPT, D), jnp.float32),
            pltpu.VMEM((SB,), jnp.int32),
            pltpu.VMEM((SB + LANES,), jnp.int32),
            pltpu.VMEM((SB, D), jnp.float32),
            pltpu.SemaphoreType.DMA,
        ],
    )
    def k(h2_hbm, eid_hbm, dst_hbm, out_hbm, acc, eidb, dstb, rows, sem):
        wid = _worker_id()
        lo = wid * NPT
        neg = jnp.full((LANES,), -jnp.inf, dtype=jnp.float32)

        def init_acc(t, carry):
            acc[t // 8, pl.ds((t % 8) * LANES, LANES)] = neg
            return carry

        lax.fori_loop(0, NPT * (D // LANES), init_acc, 0)

        def batch(b, go_prev):
            f = b // BPF
            j = b % BPF
            go = jnp.where(j == 0, jnp.int32(1), go_prev)

            @pl.when(go > 0)
            def _():
                base = pl.multiple_of((f * NW + wid) * FCAP + j * SB, 8)
                pltpu.sync_copy(eid_hbm.at[pl.ds(base, SB)], eidb)
                pltpu.sync_copy(dst_hbm.at[pl.ds(base, SB)],
                                dstb.at[pl.ds(0, SB)])
                pltpu.async_copy(h2_hbm.at[eidb], rows, sem).wait()

                def row_body(r, carry2):
                    dv = dstb[pl.ds(r, LANES)][0]

                    @pl.when(dv >= 0)
                    def _():
                        l = dv - lo
                        for jj in range(D // LANES):
                            sl = pl.ds(jj * LANES, LANES)
                            acc[l, sl] = jnp.maximum(acc[l, sl], rows[r, sl])

                    return carry2

                lax.fori_loop(0, SB, row_body, 0)

            dlast = dstb[pl.ds(SB - LANES, LANES)][LANES - 1]
            return jnp.where((go > 0) & (dlast >= 0),
                             jnp.int32(1), jnp.int32(0))

        lax.fori_loop(0, NW * BPF, batch, jnp.int32(1))

        def clean(t, carry):
            i = t // 8
            sl = pl.ds((t % 8) * LANES, LANES)
            v = acc[i, sl]
            ok = (v >= -BIG) & (v <= BIG)
            acc[i, sl] = jnp.where(ok, v, jnp.float32(0.0))
            return carry

        lax.fori_loop(0, NPT * (D // LANES), clean, 0)
        pltpu.sync_copy(acc, out_hbm.at[pl.ds(lo, NPT)])

    return k(h2, eidlist, dstlist)


_PREC = lax.Precision.HIGHEST


def _tc_stats(xs, xd):
    def body(xs_ref, xd_ref, out_ref):
        i = pl.program_id(0)
        xi = xd_ref[...]
        d = xs_ref[...] - xi
        z = jnp.zeros((1, D), dtype=jnp.float32)
        blk = jnp.concatenate(
            [jnp.sum(xi, axis=0, keepdims=True),
             jnp.sum(xi * xi, axis=0, keepdims=True),
             jnp.sum(d, axis=0, keepdims=True),
             jnp.sum(d * d, axis=0, keepdims=True), z, z, z, z], axis=0)

        @pl.when(i == 0)
        def _():
            out_ref[...] = blk

        @pl.when(i > 0)
        def _():
            out_ref[...] += blk

    return pl.pallas_call(
        body,
        grid=(E // EB,),
        in_specs=[pl.BlockSpec((EB, D), lambda i: (i, 0)),
                  pl.BlockSpec((EB, D), lambda i: (i, 0))],
        out_specs=pl.BlockSpec((8, D), lambda i: (0, 0)),
        out_shape=jax.ShapeDtypeStruct((8, D), jnp.float32),
    )(xs, xd)


def _tc_mlp1(xs, xd, scales, wa, wb):
    def body(xs_ref, xd_ref, sc_ref, wa_ref, wb_ref, h1_ref, st_ref):
        i = pl.program_id(0)
        xi = xd_ref[...]
        d = xs_ref[...] - xi
        a = jnp.maximum(xi * sc_ref[0:1, :] + sc_ref[1:2, :], 0.0)
        b = jnp.maximum(d * sc_ref[2:3, :] + sc_ref[3:4, :], 0.0)
        h1 = (jnp.dot(a, wa_ref[...], preferred_element_type=jnp.float32,
                      precision=_PREC)
              + jnp.dot(b, wb_ref[...], preferred_element_type=jnp.float32,
                        precision=_PREC))
        h1_ref[...] = h1
        z = jnp.zeros((1, D), dtype=jnp.float32)
        blk = jnp.concatenate(
            [jnp.sum(h1, axis=0, keepdims=True),
             jnp.sum(h1 * h1, axis=0, keepdims=True), z, z, z, z, z, z],
            axis=0)

        @pl.when(i == 0)
        def _():
            st_ref[...] = blk

        @pl.when(i > 0)
        def _():
            st_ref[...] += blk

    return pl.pallas_call(
        body,
        grid=(E // EB,),
        in_specs=[pl.BlockSpec((EB, D), lambda i: (i, 0)),
                  pl.BlockSpec((EB, D), lambda i: (i, 0)),
                  pl.BlockSpec((8, D), lambda i: (0, 0)),
                  pl.BlockSpec((D, D), lambda i: (0, 0)),
                  pl.BlockSpec((D, D), lambda i: (0, 0))],
        out_specs=[pl.BlockSpec((EB, D), lambda i: (i, 0)),
                   pl.BlockSpec((8, D), lambda i: (0, 0))],
        out_shape=[jax.ShapeDtypeStruct((E, D), jnp.float32),
                   jax.ShapeDtypeStruct((8, D), jnp.float32)],
    )(xs, xd, scales, wa, wb)


def _tc_mlp2(h1, scales, w2):
    def body(h1_ref, sc_ref, w_ref, h2_ref):
        a = jnp.maximum(h1_ref[...] * sc_ref[0:1, :] + sc_ref[1:2, :], 0.0)
        h2_ref[...] = jnp.dot(a, w_ref[...],
                              preferred_element_type=jnp.float32,
                              precision=_PREC)

    return pl.pallas_call(
        body,
        grid=(E // EB,),
        in_specs=[pl.BlockSpec((EB, D), lambda i: (i, 0)),
                  pl.BlockSpec((8, D), lambda i: (0, 0)),
                  pl.BlockSpec((D, D), lambda i: (0, 0))],
        out_specs=pl.BlockSpec((EB, D), lambda i: (i, 0)),
        out_shape=jax.ShapeDtypeStruct((E, D), jnp.float32),
    )(h1, scales, w2)


def _tc_matmul_bias(a, w, bias):
    K = a.shape[1]

    def body(a_ref, w_ref, b_ref, o_ref):
        o_ref[...] = jnp.dot(a_ref[...], w_ref[...],
                             preferred_element_type=jnp.float32,
                             precision=_PREC) + b_ref[...]

    return pl.pallas_call(
        body,
        grid=(N // NB,),
        in_specs=[pl.BlockSpec((NB, K), lambda i: (i, 0)),
                  pl.BlockSpec((K, D), lambda i: (0, 0)),
                  pl.BlockSpec((1, D), lambda i: (0, 0))],
        out_specs=pl.BlockSpec((NB, D), lambda i: (i, 0)),
        out_shape=jax.ShapeDtypeStruct((N, D), jnp.float32),
    )(a, w, bias)


def _bn_fold(s, ss, g, b):
    mean = s / E
    var = ss / E - mean * mean
    scale = g / jnp.sqrt(var + EPS)
    return scale, b - mean * scale


def _pack8(rows):
    z = jnp.zeros((8 - len(rows), D), dtype=jnp.float32)
    return jnp.concatenate([jnp.stack(rows, axis=0), z], axis=0)


def _edge_conv(g, src, dst, eidlist, dstlist, g1, b1, W1, g2, b2, W2):
    xs, xd = _sc_gather(g, src, dst)
    st = _tc_stats(xs, xd)
    sa, ta = _bn_fold(st[0], st[1], g1[:D], b1[:D])
    sb, tb = _bn_fold(st[2], st[3], g1[D:], b1[D:])
    h1, st1 = _tc_mlp1(xs, xd, _pack8([sa, ta, sb, tb]),
                       W1[:, :D].T, W1[:, D:].T)
    s2, t2 = _bn_fold(st1[0], st1[1], g2, b2)
    h2 = _tc_mlp2(h1, _pack8([s2, t2]), W2.T)
    return _sc_scatter_max(h2, eidlist, dstlist)[:N]


def kernel(x, edge_index, Wav, bav,
           l1_g1, l1_b1, l1_W1, l1_g2, l1_b2, l1_W2,
           l2_g1, l2_b1, l2_W1, l2_g2, l2_b2, l2_W2,
           l3_g1, l3_b1, l3_W1, l3_g2, l3_b2, l3_W2,
           l4_g1, l4_b1, l4_W1, l4_g2, l4_b2, l4_W2,
           Wout, bout):
    src = edge_index[0]
    dst = edge_index[1]
    dst_pad = jnp.concatenate([dst, jnp.full((24,), -1, jnp.int32)])
    ramp = jnp.arange(E + 24, dtype=jnp.int32)
    eidlist, dstlist = _sc_bucket(dst_pad, ramp)
    gf = _tc_matmul_bias(x.reshape(N, 2 * D), Wav.T, bav.reshape(1, D))

    g1 = _edge_conv(gf, src, dst, eidlist, dstlist,
                    l1_g1, l1_b1, l1_W1, l1_g2, l1_b2, l1_W2)
    g2 = _edge_conv(g1, src, dst, eidlist, dstlist,
                    l2_g1, l2_b1, l2_W1, l2_g2, l2_b2, l2_W2) + g1
    g3 = _edge_conv(g2, src, dst, eidlist, dstlist,
                    l3_g1, l3_b1, l3_W1, l3_g2, l3_b2, l3_W2) + g2
    g4 = _edge_conv(g3, src, dst, eidlist, dstlist,
                    l4_g1, l4_b1, l4_W1, l4_g2, l4_b2, l4_W2) + g3

    wout_pad = jnp.zeros((D, D), jnp.float32).at[:, :2].set(Wout.T)
    bout_pad = jnp.zeros((1, D), jnp.float32).at[0, :2].set(bout)
    return _tc_matmul_bias(g4, wout_pad, bout_pad)[:, :2]


# speedup vs baseline: 14.4819x; 14.4819x over previous
import functools

import jax
import jax.numpy as jnp
from jax import lax
from jax.experimental import pallas as pl
from jax.experimental.pallas import tpu as pltpu
from jax.experimental.pallas import tpu_sc as plsc

N = 10000
E = 160000
D = 128
EPS = 1e-5

NC = 2
NS = 16
NW = NC * NS
LANES = 16

EPW = E // NW
GCH = 1000

NPT = 320
NPAD = NPT * NW
BIG = 3.0e38

EB = 4000
NB = 1000


_MESH = dict(core_axis_name="c", subcore_axis_name="s")


def _worker_id():
    return lax.axis_index("s") * NC + lax.axis_index("c")


def _sc_gather(x, src, dst):
    @functools.partial(
        pl.kernel,
        mesh=plsc.VectorSubcoreMesh(**_MESH),
        out_type=[
            jax.ShapeDtypeStruct((E, D), jnp.float32),
            jax.ShapeDtypeStruct((E, D), jnp.float32),
        ],
        scratch_types=[
            pltpu.VMEM((GCH,), jnp.int32),
            pltpu.VMEM((GCH, D), jnp.float32),
            pltpu.SemaphoreType.DMA,
        ],
    )
    def k(x_hbm, src_hbm, dst_hbm, xs_out, xd_out, idx_v, rows_v, sem):
        base = _worker_id() * EPW

        def body(c, carry):
            off = base + c * GCH
            pltpu.sync_copy(src_hbm.at[pl.ds(off, GCH)], idx_v)
            pltpu.async_copy(x_hbm.at[idx_v], rows_v, sem).wait()
            pltpu.sync_copy(rows_v, xs_out.at[pl.ds(off, GCH)])
            pltpu.sync_copy(dst_hbm.at[pl.ds(off, GCH)], idx_v)
            pltpu.async_copy(x_hbm.at[idx_v], rows_v, sem).wait()
            pltpu.sync_copy(rows_v, xd_out.at[pl.ds(off, GCH)])
            return carry

        lax.fori_loop(0, EPW // GCH, body, 0)

    return k(x, src, dst)


FB = 528
FCAP = 5120
LISTSZ = NW * NW * FCAP
SB = 256
BPF = FCAP // SB


def _sc_bucket(dst_pad, ramp):
    @functools.partial(
        pl.kernel,
        mesh=plsc.VectorSubcoreMesh(**_MESH),
        out_type=[
            jax.ShapeDtypeStruct((LISTSZ,), jnp.int32),
            jax.ShapeDtypeStruct((LISTSZ,), jnp.int32),
        ],
        scratch_types=[
            pltpu.VMEM((EPW + 24,), jnp.int32),
            pltpu.VMEM((EPW + 24,), jnp.int32),
            pltpu.VMEM((NW * FB,), jnp.int32),
            pltpu.VMEM((NW * FB,), jnp.int32),
            pltpu.SMEM((NW,), jnp.int32),
            pltpu.SMEM((NW,), jnp.int32),
        ],
    )
    def k(dst_hbm, ramp_hbm, eidlist, dstlist, dstv, rampv, fbe, fbd,
          cnts, flushed):
        wid = _worker_id()
        sbase = wid * EPW
        iota = lax.iota(jnp.int32, LANES)

        def init_cnt(i, carry):
            cnts[i] = jnp.int32(0)
            flushed[i] = jnp.int32(0)
            return carry

        lax.fori_loop(0, NW, init_cnt, 0)

        pltpu.sync_copy(dst_hbm.at[pl.ds(sbase, EPW + 24)], dstv)
        pltpu.sync_copy(ramp_hbm.at[pl.ds(sbase, EPW + 24)], rampv)

        def lane(v, kk):
            d16 = dstv[pl.ds(v * LANES, LANES)]
            dk = d16[kk]
            ow = dk // NPT
            pos = cnts[ow]
            fb0 = ow * FB
            w = pl.ds(v * LANES + kk, LANES)
            fbe[pl.ds(fb0 + pos, LANES)] = rampv[w]
            fbd[pl.ds(fb0 + pos, LANES)] = dstv[w]
            cnts[ow] = pos + 1

            @pl.when(pos + 1 >= 512)
            def _():
                base = pl.multiple_of(
                    (wid * NW + ow) * FCAP + flushed[ow], 8)
                src = pl.multiple_of(fb0, 8)
                pltpu.sync_copy(fbe.at[pl.ds(src, 512)],
                                eidlist.at[pl.ds(base, 512)])
                pltpu.sync_copy(fbd.at[pl.ds(src, 512)],
                                dstlist.at[pl.ds(base, 512)])
                fbe[pl.ds(fb0, LANES)] = fbe[pl.ds(fb0 + 512, LANES)]
                fbd[pl.ds(fb0, LANES)] = fbd[pl.ds(fb0 + 512, LANES)]
                cnts[ow] = pos + 1 - 512
                flushed[ow] = flushed[ow] + 512

        def vreg(v, carry):
            for kk in range(LANES):
                lane(v, kk)
            return carry

        lax.fori_loop(0, EPW // LANES, vreg, 0)
        for kk in range(EPW - (EPW // LANES) * LANES):
            lane(EPW // LANES, kk)

        def finflush(ow, carry):
            p = cnts[ow]
            fb0 = ow * FB
            for i in range(512 // LANES):
                sl = pl.ds(fb0 + i * LANES, LANES)
                keep = (iota + (i * LANES)) < p
                fbe[sl] = jnp.where(keep, fbe[sl], iota + (i * LANES))
                fbd[sl] = jnp.where(keep, fbd[sl], -1)
            base = pl.multiple_of((wid * NW + ow) * FCAP + flushed[ow], 8)
            src = pl.multiple_of(fb0, 8)
            pltpu.sync_copy(fbe.at[pl.ds(src, 512)],
                            eidlist.at[pl.ds(base, 512)])
            pltpu.sync_copy(fbd.at[pl.ds(src, 512)],
                            dstlist.at[pl.ds(base, 512)])
            return carry

        lax.fori_loop(0, NW, finflush, 0)

    return k(dst_pad, ramp)


def _sc_scatter_max(h2, eidlist, dstlist):
    @functools.partial(
        pl.kernel,
        mesh=plsc.VectorSubcoreMesh(**_MESH),
        out_type=jax.ShapeDtypeStruct((NPAD, D), jnp.float32),
        scratch_types=[
            pltpu.VMEM((NPT, D), jnp.float32),
            pltpu.VMEM((SB,), jnp.int32),
            pltpu.VMEM((SB + LANES,), jnp.int32),
            pltpu.VMEM((SB, D), jnp.float32),
            pltpu.SemaphoreType.DMA,
        ],
    )
    def k(h2_hbm, eid_hbm, dst_hbm, out_hbm, acc, eidb, dstb, rows, sem):
        wid = _worker_id()
        lo = wid * NPT
        neg = jnp.full((LANES,), -jnp.inf, dtype=jnp.float32)

        def init_acc(t, carry):
            acc[t // 8, pl.ds((t % 8) * LANES, LANES)] = neg
            return carry

        lax.fori_loop(0, NPT * (D // LANES), init_acc, 0)

        def batch(b, go_prev):
            f = b // BPF
            j = b % BPF
            go = jnp.where(j == 0, jnp.int32(1), go_prev)

            @pl.when(go > 0)
            def _():
                base = pl.multiple_of((f * NW + wid) * FCAP + j * SB, 8)
                pltpu.sync_copy(dst_hbm.at[pl.ds(base, SB)],
                                dstb.at[pl.ds(0, SB)])
                d0 = dstb[pl.ds(0, LANES)][0]

                @pl.when(d0 >= 0)
                def _():
                    pltpu.sync_copy(eid_hbm.at[pl.ds(base, SB)], eidb)
                    pltpu.async_copy(h2_hbm.at[eidb], rows, sem).wait()

                    def row_body(r, carry2):
                        dv = dstb[pl.ds(r, LANES)][0]

                        @pl.when(dv >= 0)
                        def _():
                            l = dv - lo
                            for jj in range(D // LANES):
                                sl = pl.ds(jj * LANES, LANES)
                                acc[l, sl] = jnp.maximum(acc[l, sl],
                                                         rows[r, sl])

                        return carry2

                    lax.fori_loop(0, SB, row_body, 0)

            dlast = dstb[pl.ds(SB - LANES, LANES)][LANES - 1]
            d0b = dstb[pl.ds(0, LANES)][0]
            return jnp.where((go > 0) & (d0b >= 0) & (dlast >= 0),
                             jnp.int32(1), jnp.int32(0))

        lax.fori_loop(0, NW * BPF, batch, jnp.int32(1))

        def clean(t, carry):
            i = t // 8
            sl = pl.ds((t % 8) * LANES, LANES)
            v = acc[i, sl]
            ok = (v >= -BIG) & (v <= BIG)
            acc[i, sl] = jnp.where(ok, v, jnp.float32(0.0))
            return carry

        lax.fori_loop(0, NPT * (D // LANES), clean, 0)
        pltpu.sync_copy(acc, out_hbm.at[pl.ds(lo, NPT)])

    return k(h2, eidlist, dstlist)


_PREC = lax.Precision.HIGHEST


def _tc_stats(xs, xd):
    def body(xs_ref, xd_ref, out_ref):
        i = pl.program_id(0)
        xi = xd_ref[...]
        d = xs_ref[...] - xi
        z = jnp.zeros((1, D), dtype=jnp.float32)
        blk = jnp.concatenate(
            [jnp.sum(xi, axis=0, keepdims=True),
             jnp.sum(xi * xi, axis=0, keepdims=True),
             jnp.sum(d, axis=0, keepdims=True),
             jnp.sum(d * d, axis=0, keepdims=True), z, z, z, z], axis=0)

        @pl.when(i == 0)
        def _():
            out_ref[...] = blk

        @pl.when(i > 0)
        def _():
            out_ref[...] += blk

    return pl.pallas_call(
        body,
        grid=(E // EB,),
        in_specs=[pl.BlockSpec((EB, D), lambda i: (i, 0)),
                  pl.BlockSpec((EB, D), lambda i: (i, 0))],
        out_specs=pl.BlockSpec((8, D), lambda i: (0, 0)),
        out_shape=jax.ShapeDtypeStruct((8, D), jnp.float32),
    )(xs, xd)


def _tc_mlp1(xs, xd, scales, wa, wb):
    def body(xs_ref, xd_ref, sc_ref, wa_ref, wb_ref, h1_ref, st_ref):
        i = pl.program_id(0)
        xi = xd_ref[...]
        d = xs_ref[...] - xi
        a = jnp.maximum(xi * sc_ref[0:1, :] + sc_ref[1:2, :], 0.0)
        b = jnp.maximum(d * sc_ref[2:3, :] + sc_ref[3:4, :], 0.0)
        h1 = (jnp.dot(a, wa_ref[...], preferred_element_type=jnp.float32,
                      precision=_PREC)
              + jnp.dot(b, wb_ref[...], preferred_element_type=jnp.float32,
                        precision=_PREC))
        h1_ref[...] = h1
        z = jnp.zeros((1, D), dtype=jnp.float32)
        blk = jnp.concatenate(
            [jnp.sum(h1, axis=0, keepdims=True),
             jnp.sum(h1 * h1, axis=0, keepdims=True), z, z, z, z, z, z],
            axis=0)

        @pl.when(i == 0)
        def _():
            st_ref[...] = blk

        @pl.when(i > 0)
        def _():
            st_ref[...] += blk

    return pl.pallas_call(
        body,
        grid=(E // EB,),
        in_specs=[pl.BlockSpec((EB, D), lambda i: (i, 0)),
                  pl.BlockSpec((EB, D), lambda i: (i, 0)),
                  pl.BlockSpec((8, D), lambda i: (0, 0)),
                  pl.BlockSpec((D, D), lambda i: (0, 0)),
                  pl.BlockSpec((D, D), lambda i: (0, 0))],
        out_specs=[pl.BlockSpec((EB, D), lambda i: (i, 0)),
                   pl.BlockSpec((8, D), lambda i: (0, 0))],
        out_shape=[jax.ShapeDtypeStruct((E, D), jnp.float32),
                   jax.ShapeDtypeStruct((8, D), jnp.float32)],
    )(xs, xd, scales, wa, wb)


def _tc_mlp2(h1, scales, w2):
    def body(h1_ref, sc_ref, w_ref, h2_ref):
        a = jnp.maximum(h1_ref[...] * sc_ref[0:1, :] + sc_ref[1:2, :], 0.0)
        h2_ref[...] = jnp.dot(a, w_ref[...],
                              preferred_element_type=jnp.float32,
                              precision=_PREC)

    return pl.pallas_call(
        body,
        grid=(E // EB,),
        in_specs=[pl.BlockSpec((EB, D), lambda i: (i, 0)),
                  pl.BlockSpec((8, D), lambda i: (0, 0)),
                  pl.BlockSpec((D, D), lambda i: (0, 0))],
        out_specs=pl.BlockSpec((EB, D), lambda i: (i, 0)),
        out_shape=jax.ShapeDtypeStruct((E, D), jnp.float32),
    )(h1, scales, w2)


def _tc_matmul_bias(a, w, bias):
    K = a.shape[1]

    def body(a_ref, w_ref, b_ref, o_ref):
        o_ref[...] = jnp.dot(a_ref[...], w_ref[...],
                             preferred_element_type=jnp.float32,
                             precision=_PREC) + b_ref[...]

    return pl.pallas_call(
        body,
        grid=(N // NB,),
        in_specs=[pl.BlockSpec((NB, K), lambda i: (i, 0)),
                  pl.BlockSpec((K, D), lambda i: (0, 0)),
                  pl.BlockSpec((1, D), lambda i: (0, 0))],
        out_specs=pl.BlockSpec((NB, D), lambda i: (i, 0)),
        out_shape=jax.ShapeDtypeStruct((N, D), jnp.float32),
    )(a, w, bias)


def _bn_fold(s, ss, g, b):
    mean = s / E
    var = ss / E - mean * mean
    scale = g / jnp.sqrt(var + EPS)
    return scale, b - mean * scale


def _pack8(rows):
    z = jnp.zeros((8 - len(rows), D), dtype=jnp.float32)
    return jnp.concatenate([jnp.stack(rows, axis=0), z], axis=0)


def _edge_conv(g, src, dst, eidlist, dstlist, g1, b1, W1, g2, b2, W2):
    xs, xd = _sc_gather(g, src, dst)
    st = _tc_stats(xs, xd)
    sa, ta = _bn_fold(st[0], st[1], g1[:D], b1[:D])
    sb, tb = _bn_fold(st[2], st[3], g1[D:], b1[D:])
    h1, st1 = _tc_mlp1(xs, xd, _pack8([sa, ta, sb, tb]),
                       W1[:, :D].T, W1[:, D:].T)
    s2, t2 = _bn_fold(st1[0], st1[1], g2, b2)
    h2 = _tc_mlp2(h1, _pack8([s2, t2]), W2.T)
    return _sc_scatter_max(h2, eidlist, dstlist)[:N]


def kernel(x, edge_index, Wav, bav,
           l1_g1, l1_b1, l1_W1, l1_g2, l1_b2, l1_W2,
           l2_g1, l2_b1, l2_W1, l2_g2, l2_b2, l2_W2,
           l3_g1, l3_b1, l3_W1, l3_g2, l3_b2, l3_W2,
           l4_g1, l4_b1, l4_W1, l4_g2, l4_b2, l4_W2,
           Wout, bout):
    src = edge_index[0]
    dst = edge_index[1]
    dst_pad = jnp.concatenate([dst, jnp.full((24,), -1, jnp.int32)])
    ramp = jnp.arange(E + 24, dtype=jnp.int32)
    eidlist, dstlist = _sc_bucket(dst_pad, ramp)
    gf = _tc_matmul_bias(x.reshape(N, 2 * D), Wav.T, bav.reshape(1, D))

    g1 = _edge_conv(gf, src, dst, eidlist, dstlist,
                    l1_g1, l1_b1, l1_W1, l1_g2, l1_b2, l1_W2)
    g2 = _edge_conv(g1, src, dst, eidlist, dstlist,
                    l2_g1, l2_b1, l2_W1, l2_g2, l2_b2, l2_W2) + g1
    g3 = _edge_conv(g2, src, dst, eidlist, dstlist,
                    l3_g1, l3_b1, l3_W1, l3_g2, l3_b2, l3_W2) + g2
    g4 = _edge_conv(g3, src, dst, eidlist, dstlist,
                    l4_g1, l4_b1, l4_W1, l4_g2, l4_b2, l4_W2) + g3

    wout_pad = jnp.zeros((D, D), jnp.float32).at[:, :2].set(Wout.T)
    bout_pad = jnp.zeros((1, D), jnp.float32).at[0, :2].set(bout)
    return _tc_matmul_bias(g4, wout_pad, bout_pad)[:, :2]
